# Initial kernel scaffold; baseline (speedup 1.0000x reference)
import functools

import jax
import jax.numpy as jnp
from jax import lax
from jax.experimental import pallas as pl
from jax.experimental.pallas import tpu as pltpu
from jax.experimental.pallas import tpu_sc as plsc

N = 10000
E = 320000
H = 128
L = 3
G = 128

NC = 2
NS = 16
LANES = 16
NW = NC * NS
EPW = E // NW
CHUNK = 80
NCHUNK = EPW // CHUNK
RPT = N // NS

_f32 = jnp.float32


def _edge_stage_body(h_hbm, e_hbm, src_hbm, dst_hbm, zeros_hbm, out_hbm,
                     sidx_v, didx_v, rows_v, er_v, aggr_sh):
    cid = lax.axis_index("c")
    sid = lax.axis_index("s")
    wid = cid * NS + sid

    pltpu.sync_copy(zeros_hbm.at[pl.ds(sid * RPT, RPT)],
                    aggr_sh.at[pl.ds(sid * RPT, RPT)])
    plsc.subcore_barrier()

    base0 = wid * EPW

    @pl.loop(0, NCHUNK)
    def _(ci):
        base = base0 + ci * CHUNK
        pltpu.sync_copy(src_hbm.at[pl.ds(base, CHUNK)], sidx_v)
        pltpu.sync_copy(dst_hbm.at[pl.ds(base, CHUNK)], didx_v)
        pltpu.sync_copy(h_hbm.at[sidx_v], rows_v)
        pltpu.sync_copy(e_hbm.at[pl.ds(base, CHUNK)], er_v)

        @pl.loop(0, CHUNK)
        def _(r):
            for c in range(0, H, LANES):
                slc = (pl.ds(r, 1), pl.ds(c, LANES))
                v = rows_v.at[*slc][...] + er_v.at[*slc][...]
                rows_v.at[*slc][...] = jnp.maximum(v, 0.0)

        pltpu.sync_copy(rows_v, aggr_sh.at[didx_v], add=True)

    plsc.subcore_barrier()
    pltpu.sync_copy(aggr_sh.at[pl.ds(sid * RPT, RPT)],
                    out_hbm.at[cid, pl.ds(sid * RPT, RPT)])


def _edge_stage(h, e, src, dst, zeros):
    mesh = plsc.VectorSubcoreMesh(core_axis_name="c", subcore_axis_name="s",
                                  num_cores=NC, num_subcores=NS)
    k = pl.kernel(
        _edge_stage_body,
        out_type=jax.ShapeDtypeStruct((NC, N, H), _f32),
        mesh=mesh,
        scratch_types=[
            pltpu.VMEM((CHUNK,), jnp.int32),
            pltpu.VMEM((CHUNK,), jnp.int32),
            pltpu.VMEM((CHUNK, H), _f32),
            pltpu.VMEM((CHUNK, H), _f32),
            pltpu.VMEM_SHARED((N, H), _f32),
        ],
    )
    return k(h, e, src, dst, zeros)


def _node_encode_body(x_ref, w_ref, b_ref, o_ref):
    o_ref[...] = (jnp.dot(x_ref[...], w_ref[...],
                          preferred_element_type=_f32) + b_ref[...])


def _edge_encode_body(ea_ref, w1_ref, b1_ref, w2_ref, b2_ref, o_ref):
    t = jnp.dot(ea_ref[...], w1_ref[...], preferred_element_type=_f32)
    t = jnp.maximum(t + b1_ref[...], 0.0)
    o_ref[...] = (jnp.dot(t, w2_ref[...], preferred_element_type=_f32)
                  + b2_ref[...])


def _node_mlp_body(h_ref, a0_ref, a1_ref, w1_ref, b1_ref, w2_ref, b2_ref,
                   o_ref):
    z = h_ref[...] + a0_ref[...] + a1_ref[...]
    t = jnp.dot(z, w1_ref[...], preferred_element_type=_f32)
    t = jnp.maximum(t + b1_ref[...], 0.0)
    t = jnp.dot(t, w2_ref[...], preferred_element_type=_f32) + b2_ref[...]
    o_ref[...] = jnp.maximum(t, 0.0)


def _pool_heads_body(h_ref, batch_ref, cw1_ref, cb1_ref, cw2_ref, cb2_ref,
                     rw1_ref, rb1_ref, rw2_ref, rb2_ref, s_ref, r_ref):
    b2d = batch_ref[...]
    gids = lax.broadcasted_iota(jnp.int32, (N, G), 1)
    p = (b2d == gids).astype(_f32)
    dims = (((0,), (0,)), ((), ()))
    sums = lax.dot_general(p, h_ref[...], dims,
                           preferred_element_type=_f32)
    counts = lax.dot_general(p, jnp.ones((N, 1), _f32), dims,
                             preferred_element_type=_f32)
    g = sums / jnp.maximum(counts, 1.0)
    cs = jnp.maximum(jnp.dot(g, cw1_ref[...], preferred_element_type=_f32)
                     + cb1_ref[...], 0.0)
    s_ref[...] = (jnp.dot(cs, cw2_ref[...], preferred_element_type=_f32)
                  + cb2_ref[...])
    rs = jnp.maximum(jnp.dot(g, rw1_ref[...], preferred_element_type=_f32)
                     + rb1_ref[...], 0.0)
    r_ref[...] = (jnp.dot(rs, rw2_ref[...], preferred_element_type=_f32)
                  + rb2_ref[...])


def _node_encode(x, w, b):
    return pl.pallas_call(
        _node_encode_body,
        out_shape=jax.ShapeDtypeStruct((N, H), _f32),
    )(x, w, b)


_BE = 2560


def _edge_encode(ea, w1, b1, w2, b2):
    nblk = E // _BE
    return pl.pallas_call(
        _edge_encode_body,
        grid=(nblk,),
        in_specs=[
            pl.BlockSpec((_BE, 3), lambda i: (i, 0)),
            pl.BlockSpec((3, H), lambda i: (0, 0)),
            pl.BlockSpec((1, H), lambda i: (0, 0)),
            pl.BlockSpec((H, H), lambda i: (0, 0)),
            pl.BlockSpec((1, H), lambda i: (0, 0)),
        ],
        out_specs=pl.BlockSpec((_BE, H), lambda i: (i, 0)),
        out_shape=jax.ShapeDtypeStruct((E, H), _f32),
    )(ea, w1, b1, w2, b2)


_BN = 2000


def _node_mlp(h, a0, a1, w1, b1, w2, b2):
    nblk = N // _BN
    return pl.pallas_call(
        _node_mlp_body,
        grid=(nblk,),
        in_specs=[
            pl.BlockSpec((_BN, H), lambda i: (i, 0)),
            pl.BlockSpec((_BN, H), lambda i: (i, 0)),
            pl.BlockSpec((_BN, H), lambda i: (i, 0)),
            pl.BlockSpec((H, H), lambda i: (0, 0)),
            pl.BlockSpec((1, H), lambda i: (0, 0)),
            pl.BlockSpec((H, H), lambda i: (0, 0)),
            pl.BlockSpec((1, H), lambda i: (0, 0)),
        ],
        out_specs=pl.BlockSpec((_BN, H), lambda i: (i, 0)),
        out_shape=jax.ShapeDtypeStruct((N, H), _f32),
    )(h, a0, a1, w1, b1, w2, b2)


def _pool_heads(h, batch2d, cw1, cb1, cw2, cb2, rw1, rb1, rw2, rb2):
    return pl.pallas_call(
        _pool_heads_body,
        out_shape=(jax.ShapeDtypeStruct((G, 1), _f32),
                   jax.ShapeDtypeStruct((G, 1), _f32)),
    )(h, batch2d, cw1, cb1, cw2, cb2, rw1, rb1, rw2, rb2)


def kernel(x, edge_index, edge_attr, batch,
           ne_w, ne_b, ee_w1, ee_b1, ee_w2, ee_b2,
           conv_w1, conv_b1, conv_w2, conv_b2,
           ch_w1, ch_b1, ch_w2, ch_b2,
           rh_w1, rh_b1, rh_w2, rh_b2):
    src = edge_index[0]
    dst = edge_index[1]
    zeros = jnp.zeros((N, H), _f32)

    h = _node_encode(x, ne_w, ne_b.reshape(1, H))
    e = _edge_encode(edge_attr, ee_w1, ee_b1.reshape(1, H),
                     ee_w2, ee_b2.reshape(1, H))

    for i in range(L):
        parts = _edge_stage(h, e, src, dst, zeros)
        h = _node_mlp(h, parts[0], parts[1],
                      conv_w1[i], conv_b1[i].reshape(1, H),
                      conv_w2[i], conv_b2[i].reshape(1, H))

    s, r = _pool_heads(h, batch.reshape(N, 1),
                       ch_w1, ch_b1.reshape(1, 64), ch_w2, ch_b2.reshape(1, 1),
                       rh_w1, rh_b1.reshape(1, 64), rh_w2, rh_b2.reshape(1, 1))
    return (s.reshape(G), r.reshape(G))


# speedup vs baseline: 2.6705x; 2.6705x over previous
import functools

import jax
import jax.numpy as jnp
from jax import lax
from jax.experimental import pallas as pl
from jax.experimental.pallas import tpu as pltpu
from jax.experimental.pallas import tpu_sc as plsc

N = 10000
E = 320000
H = 128
L = 3
G = 128

NC = 2
NS = 16
LANES = 16
NW = NC * NS
EPW = E // NW
CHUNK = 80
NCHUNK = EPW // CHUNK
NPAD = 10240
RPT = NPAD // NS

_f32 = jnp.float32


def _edge_stage_body(h_hbm, e_hbm, src_hbm, dst_hbm, zeros_hbm, out_hbm,
                     sidx_v, didx_v, rows_v, er_v, aggr_sh):
    cid = lax.axis_index("c")
    sid = lax.axis_index("s")
    wid = cid * NS + sid

    pltpu.sync_copy(zeros_hbm.at[pl.ds(sid * RPT, RPT)],
                    aggr_sh.at[pl.ds(sid * RPT, RPT)])
    plsc.subcore_barrier()

    base0 = wid * EPW

    @pl.loop(0, NCHUNK)
    def _(ci):
        base = base0 + ci * CHUNK
        pltpu.sync_copy(src_hbm.at[pl.ds(base, CHUNK)], sidx_v)
        pltpu.sync_copy(dst_hbm.at[pl.ds(base, CHUNK)], didx_v)
        pltpu.sync_copy(h_hbm.at[sidx_v], rows_v)
        pltpu.sync_copy(e_hbm.at[pl.ds(base, CHUNK)], er_v)

        @pl.loop(0, CHUNK)
        def _(r):
            for c in range(0, H, LANES):
                slc = (pl.ds(r, 1), pl.ds(c, LANES))
                v = rows_v.at[*slc][...] + er_v.at[*slc][...]
                rows_v.at[*slc][...] = jnp.maximum(v, 0.0)

        pltpu.sync_copy(rows_v, aggr_sh.at[didx_v], add=True)

    plsc.subcore_barrier()
    pltpu.sync_copy(aggr_sh.at[pl.ds(sid * RPT, RPT)],
                    out_hbm.at[cid, pl.ds(sid * RPT, RPT)])


def _edge_stage(h, e, src, dst, zeros):
    mesh = plsc.VectorSubcoreMesh(core_axis_name="c", subcore_axis_name="s",
                                  num_cores=NC, num_subcores=NS)
    k = pl.kernel(
        _edge_stage_body,
        out_type=jax.ShapeDtypeStruct((NC, NPAD, H), _f32),
        mesh=mesh,
        scratch_types=[
            pltpu.VMEM((CHUNK,), jnp.int32),
            pltpu.VMEM((CHUNK,), jnp.int32),
            pltpu.VMEM((CHUNK, H), _f32),
            pltpu.VMEM((CHUNK, H), _f32),
            pltpu.VMEM_SHARED((NPAD, H), _f32),
        ],
    )
    return k(h, e, src, dst, zeros)


def _node_encode_body(x_ref, w_ref, b_ref, o_ref):
    o_ref[...] = (jnp.dot(x_ref[...], w_ref[...],
                          preferred_element_type=_f32) + b_ref[...])


def _edge_encode_body(ea_ref, w1_ref, b1_ref, w2_ref, b2_ref, o_ref):
    t = jnp.dot(ea_ref[...], w1_ref[...], preferred_element_type=_f32)
    t = jnp.maximum(t + b1_ref[...], 0.0)
    o_ref[...] = (jnp.dot(t, w2_ref[...], preferred_element_type=_f32)
                  + b2_ref[...])


def _node_mlp_body(h_ref, a0_ref, a1_ref, w1_ref, b1_ref, w2_ref, b2_ref,
                   o_ref):
    z = h_ref[...] + a0_ref[...] + a1_ref[...]
    t = jnp.dot(z, w1_ref[...], preferred_element_type=_f32)
    t = jnp.maximum(t + b1_ref[...], 0.0)
    t = jnp.dot(t, w2_ref[...], preferred_element_type=_f32) + b2_ref[...]
    o_ref[...] = jnp.maximum(t, 0.0)


def _pool_heads_body(h_ref, batch_ref, cw1_ref, cb1_ref, cw2_ref, cb2_ref,
                     rw1_ref, rb1_ref, rw2_ref, rb2_ref, s_ref, r_ref):
    b2d = batch_ref[...]
    gids = lax.broadcasted_iota(jnp.int32, (N, G), 1)
    p = (b2d == gids).astype(_f32)
    dims = (((0,), (0,)), ((), ()))
    sums = lax.dot_general(p, h_ref[...], dims,
                           preferred_element_type=_f32)
    counts = lax.dot_general(p, jnp.ones((N, 1), _f32), dims,
                             preferred_element_type=_f32)
    g = sums / jnp.maximum(counts, 1.0)
    cs = jnp.maximum(jnp.dot(g, cw1_ref[...], preferred_element_type=_f32)
                     + cb1_ref[...], 0.0)
    s_ref[...] = (jnp.dot(cs, cw2_ref[...], preferred_element_type=_f32)
                  + cb2_ref[...])
    rs = jnp.maximum(jnp.dot(g, rw1_ref[...], preferred_element_type=_f32)
                     + rb1_ref[...], 0.0)
    r_ref[...] = (jnp.dot(rs, rw2_ref[...], preferred_element_type=_f32)
                  + rb2_ref[...])


def _node_encode(x, w, b):
    return pl.pallas_call(
        _node_encode_body,
        out_shape=jax.ShapeDtypeStruct((N, H), _f32),
    )(x, w, b)


_BE = 2560


def _edge_encode(ea, w1, b1, w2, b2):
    nblk = E // _BE
    return pl.pallas_call(
        _edge_encode_body,
        grid=(nblk,),
        in_specs=[
            pl.BlockSpec((_BE, 3), lambda i: (i, 0)),
            pl.BlockSpec((3, H), lambda i: (0, 0)),
            pl.BlockSpec((1, H), lambda i: (0, 0)),
            pl.BlockSpec((H, H), lambda i: (0, 0)),
            pl.BlockSpec((1, H), lambda i: (0, 0)),
        ],
        out_specs=pl.BlockSpec((_BE, H), lambda i: (i, 0)),
        out_shape=jax.ShapeDtypeStruct((E, H), _f32),
    )(ea, w1, b1, w2, b2)


_BN = 2000


def _node_mlp(h, a0, a1, w1, b1, w2, b2):
    nblk = N // _BN
    return pl.pallas_call(
        _node_mlp_body,
        grid=(nblk,),
        in_specs=[
            pl.BlockSpec((_BN, H), lambda i: (i, 0)),
            pl.BlockSpec((_BN, H), lambda i: (i, 0)),
            pl.BlockSpec((_BN, H), lambda i: (i, 0)),
            pl.BlockSpec((H, H), lambda i: (0, 0)),
            pl.BlockSpec((1, H), lambda i: (0, 0)),
            pl.BlockSpec((H, H), lambda i: (0, 0)),
            pl.BlockSpec((1, H), lambda i: (0, 0)),
        ],
        out_specs=pl.BlockSpec((_BN, H), lambda i: (i, 0)),
        out_shape=jax.ShapeDtypeStruct((N, H), _f32),
    )(h, a0, a1, w1, b1, w2, b2)


def _pool_heads(h, batch2d, cw1, cb1, cw2, cb2, rw1, rb1, rw2, rb2):
    return pl.pallas_call(
        _pool_heads_body,
        out_shape=(jax.ShapeDtypeStruct((G, 1), _f32),
                   jax.ShapeDtypeStruct((G, 1), _f32)),
    )(h, batch2d, cw1, cb1, cw2, cb2, rw1, rb1, rw2, rb2)


def kernel(x, edge_index, edge_attr, batch,
           ne_w, ne_b, ee_w1, ee_b1, ee_w2, ee_b2,
           conv_w1, conv_b1, conv_w2, conv_b2,
           ch_w1, ch_b1, ch_w2, ch_b2,
           rh_w1, rh_b1, rh_w2, rh_b2):
    src = edge_index[0]
    dst = edge_index[1]
    zeros = jnp.zeros((NPAD, H), _f32)

    h = _node_encode(x, ne_w, ne_b.reshape(1, H))
    e = _edge_encode(edge_attr, ee_w1, ee_b1.reshape(1, H),
                     ee_w2, ee_b2.reshape(1, H))

    for i in range(L):
        parts = _edge_stage(h, e, src, dst, zeros)
        h = _node_mlp(h, parts[0, :N], parts[1, :N],
                      conv_w1[i], conv_b1[i].reshape(1, H),
                      conv_w2[i], conv_b2[i].reshape(1, H))

    s, r = _pool_heads(h, batch.reshape(N, 1),
                       ch_w1, ch_b1.reshape(1, 64), ch_w2, ch_b2.reshape(1, 1),
                       rh_w1, rh_b1.reshape(1, 64), rh_w2, rh_b2.reshape(1, 1))
    return (s.reshape(G), r.reshape(G))


# speedup vs baseline: 5.5421x; 2.0753x over previous
import functools

import jax
import jax.numpy as jnp
from jax import lax
from jax.experimental import pallas as pl
from jax.experimental.pallas import tpu as pltpu
from jax.experimental.pallas import tpu_sc as plsc

N = 10000
E = 320000
H = 128
L = 3
G = 128

NC = 2
NS = 16
LANES = 16
NW = NC * NS
EPW = E // NW
CHUNK = 80
NCHUNK = EPW // CHUNK
NPAD = 10240
RPT = NPAD // NS

_f32 = jnp.float32


def _edge_stage_body(h_hbm, e_hbm, src_hbm, dst_hbm, zeros_hbm, out_hbm,
                     sidx, didx, rows, sbuf, aggr_sh, *sems):
    cid = lax.axis_index("c")
    sid = lax.axis_index("s")
    wid = cid * NS + sid
    gsem = sems[0:2]
    esem = sems[2:4]
    ssem = sems[4:6]
    isem_s = sems[6:8]
    isem_d = sems[8:10]

    pltpu.sync_copy(zeros_hbm.at[pl.ds(sid * RPT, RPT)],
                    aggr_sh.at[pl.ds(sid * RPT, RPT)])

    base0 = wid * EPW

    def issue_gather_e(ci, b):
        pltpu.async_copy(h_hbm.at[sidx.at[b]], rows.at[b], gsem[b])
        pltpu.async_copy(e_hbm.at[pl.ds(base0 + ci * CHUNK, CHUNK)],
                         sbuf.at[b], esem[b])

    for b in range(2):
        pltpu.sync_copy(src_hbm.at[pl.ds(base0 + b * CHUNK, CHUNK)], sidx.at[b])
        pltpu.async_copy(dst_hbm.at[pl.ds(base0 + b * CHUNK, CHUNK)], didx.at[b], isem_d[b])
    plsc.subcore_barrier()
    for b in range(2):
        issue_gather_e(b, b)

    def step(ci, b, issue_next):
        pltpu.make_async_copy(h_hbm.at[sidx.at[b]], rows.at[b],
                              gsem[b]).wait()
        pltpu.make_async_copy(e_hbm.at[pl.ds(base0 + ci * CHUNK, CHUNK)],
                              sbuf.at[b], esem[b]).wait()

        @pl.loop(0, CHUNK)
        def _(r):
            for c in range(0, H, LANES):
                slc = (pl.ds(r, 1), pl.ds(c, LANES))
                v = sbuf.at[b].at[*slc][...] + rows.at[b].at[*slc][...]
                sbuf.at[b].at[*slc][...] = jnp.maximum(v, 0.0)

        pltpu.make_async_copy(dst_hbm.at[pl.ds(base0, CHUNK)], didx.at[b],
                              isem_d[b]).wait()
        pltpu.async_copy(sbuf.at[b], aggr_sh.at[didx.at[b]], ssem[b],
                         add=True)
        if issue_next:
            pltpu.async_copy(src_hbm.at[pl.ds(base0 + (ci + 2) * CHUNK, CHUNK)],
                             sidx.at[b], isem_s[b])
        pltpu.make_async_copy(sbuf.at[b], aggr_sh.at[didx.at[b]],
                              ssem[b]).wait()
        if issue_next:
            pltpu.async_copy(dst_hbm.at[pl.ds(base0 + (ci + 2) * CHUNK, CHUNK)],
                             didx.at[b], isem_d[b])
            pltpu.make_async_copy(src_hbm.at[pl.ds(base0, CHUNK)], sidx.at[b],
                                  isem_s[b]).wait()
            issue_gather_e(ci + 2, b)

    @pl.loop(0, NCHUNK - 3, step=2)
    def _(k):
        step(k, 0, True)
        step(k + 1, 1, True)

    step(NCHUNK - 3, 0, True)
    step(NCHUNK - 2, 1, False)
    step(NCHUNK - 1, 0, False)

    plsc.subcore_barrier()
    pltpu.sync_copy(aggr_sh.at[pl.ds(sid * RPT, RPT)],
                    out_hbm.at[cid, pl.ds(sid * RPT, RPT)])


def _edge_stage(h, e, src3, dst3, zeros):
    mesh = plsc.VectorSubcoreMesh(core_axis_name="c", subcore_axis_name="s",
                                  num_cores=NC, num_subcores=NS)
    k = pl.kernel(
        _edge_stage_body,
        out_type=jax.ShapeDtypeStruct((NC, NPAD, H), _f32),
        mesh=mesh,
        scratch_types=[
            pltpu.VMEM((2, CHUNK), jnp.int32),
            pltpu.VMEM((2, CHUNK), jnp.int32),
            pltpu.VMEM((2, CHUNK, H), _f32),
            pltpu.VMEM((2, CHUNK, H), _f32),
            pltpu.VMEM_SHARED((NPAD, H), _f32),
        ] + [pltpu.SemaphoreType.DMA] * 10,
    )
    return k(h, e, src3, dst3, zeros)


def _node_encode_body(x_ref, w_ref, b_ref, o_ref):
    o_ref[...] = (jnp.dot(x_ref[...], w_ref[...],
                          preferred_element_type=_f32) + b_ref[...])


def _edge_encode_body(ea_ref, w1_ref, b1_ref, w2_ref, b2_ref, o_ref):
    t = jnp.dot(ea_ref[...], w1_ref[...], preferred_element_type=_f32)
    t = jnp.maximum(t + b1_ref[...], 0.0)
    o_ref[...] = (jnp.dot(t, w2_ref[...], preferred_element_type=_f32)
                  + b2_ref[...])


def _node_mlp_body(h_ref, a0_ref, a1_ref, w1_ref, b1_ref, w2_ref, b2_ref,
                   o_ref):
    z = h_ref[...] + a0_ref[...] + a1_ref[...]
    t = jnp.dot(z, w1_ref[...], preferred_element_type=_f32)
    t = jnp.maximum(t + b1_ref[...], 0.0)
    t = jnp.dot(t, w2_ref[...], preferred_element_type=_f32) + b2_ref[...]
    o_ref[...] = jnp.maximum(t, 0.0)


def _pool_heads_body(h_ref, batch_ref, cw1_ref, cb1_ref, cw2_ref, cb2_ref,
                     rw1_ref, rb1_ref, rw2_ref, rb2_ref, s_ref, r_ref):
    b2d = batch_ref[...]
    gids = lax.broadcasted_iota(jnp.int32, (N, G), 1)
    p = (b2d == gids).astype(_f32)
    dims = (((0,), (0,)), ((), ()))
    sums = lax.dot_general(p, h_ref[...], dims,
                           preferred_element_type=_f32)
    counts = lax.dot_general(p, jnp.ones((N, 1), _f32), dims,
                             preferred_element_type=_f32)
    g = sums / jnp.maximum(counts, 1.0)
    cs = jnp.maximum(jnp.dot(g, cw1_ref[...], preferred_element_type=_f32)
                     + cb1_ref[...], 0.0)
    s_ref[...] = (jnp.dot(cs, cw2_ref[...], preferred_element_type=_f32)
                  + cb2_ref[...])
    rs = jnp.maximum(jnp.dot(g, rw1_ref[...], preferred_element_type=_f32)
                     + rb1_ref[...], 0.0)
    r_ref[...] = (jnp.dot(rs, rw2_ref[...], preferred_element_type=_f32)
                  + rb2_ref[...])


def _node_encode(x, w, b):
    return pl.pallas_call(
        _node_encode_body,
        out_shape=jax.ShapeDtypeStruct((N, H), _f32),
    )(x, w, b)


_BE = 2560


def _edge_encode(ea, w1, b1, w2, b2):
    nblk = E // _BE
    return pl.pallas_call(
        _edge_encode_body,
        grid=(nblk,),
        in_specs=[
            pl.BlockSpec((_BE, 3), lambda i: (i, 0)),
            pl.BlockSpec((3, H), lambda i: (0, 0)),
            pl.BlockSpec((1, H), lambda i: (0, 0)),
            pl.BlockSpec((H, H), lambda i: (0, 0)),
            pl.BlockSpec((1, H), lambda i: (0, 0)),
        ],
        out_specs=pl.BlockSpec((_BE, H), lambda i: (i, 0)),
        out_shape=jax.ShapeDtypeStruct((E, H), _f32),
    )(ea, w1, b1, w2, b2)


_BN = 2000


def _node_mlp(h, a0, a1, w1, b1, w2, b2):
    nblk = N // _BN
    return pl.pallas_call(
        _node_mlp_body,
        grid=(nblk,),
        in_specs=[
            pl.BlockSpec((_BN, H), lambda i: (i, 0)),
            pl.BlockSpec((_BN, H), lambda i: (i, 0)),
            pl.BlockSpec((_BN, H), lambda i: (i, 0)),
            pl.BlockSpec((H, H), lambda i: (0, 0)),
            pl.BlockSpec((1, H), lambda i: (0, 0)),
            pl.BlockSpec((H, H), lambda i: (0, 0)),
            pl.BlockSpec((1, H), lambda i: (0, 0)),
        ],
        out_specs=pl.BlockSpec((_BN, H), lambda i: (i, 0)),
        out_shape=jax.ShapeDtypeStruct((N, H), _f32),
    )(h, a0, a1, w1, b1, w2, b2)


def _pool_heads(h, batch2d, cw1, cb1, cw2, cb2, rw1, rb1, rw2, rb2):
    return pl.pallas_call(
        _pool_heads_body,
        out_shape=(jax.ShapeDtypeStruct((G, 1), _f32),
                   jax.ShapeDtypeStruct((G, 1), _f32)),
    )(h, batch2d, cw1, cb1, cw2, cb2, rw1, rb1, rw2, rb2)


def kernel(x, edge_index, edge_attr, batch,
           ne_w, ne_b, ee_w1, ee_b1, ee_w2, ee_b2,
           conv_w1, conv_b1, conv_w2, conv_b2,
           ch_w1, ch_b1, ch_w2, ch_b2,
           rh_w1, rh_b1, rh_w2, rh_b2):
    src = edge_index[0]
    dst = edge_index[1]
    zeros = jnp.zeros((NPAD, H), _f32)

    h = _node_encode(x, ne_w, ne_b.reshape(1, H))
    e = _edge_encode(edge_attr, ee_w1, ee_b1.reshape(1, H),
                     ee_w2, ee_b2.reshape(1, H))

    for i in range(L):
        parts = _edge_stage(h, e, src, dst, zeros)
        h = _node_mlp(h, parts[0, :N], parts[1, :N],
                      conv_w1[i], conv_b1[i].reshape(1, H),
                      conv_w2[i], conv_b2[i].reshape(1, H))

    s, r = _pool_heads(h, batch.reshape(N, 1),
                       ch_w1, ch_b1.reshape(1, 64), ch_w2, ch_b2.reshape(1, 1),
                       rh_w1, rh_b1.reshape(1, 64), rh_w2, rh_b2.reshape(1, 1))
    return (s.reshape(G), r.reshape(G))
